# baseline (speedup 1.0000x reference)
import functools
import math

import jax
import jax.numpy as jnp
import numpy as np
from jax.experimental import pallas as pl
from jax.experimental.pallas import tpu as pltpu

B = 8
V1, V2, V3 = 2048, 512, 128
NN = 20
KSEL = 21
KP = 24

TC0 = 16
T0 = 48
T1 = 80
T2 = 144
T3 = 272

_BF = jnp.bfloat16


def _f32(x):
    return x.astype(jnp.float32)


def _split3(v):
    hi = v.astype(_BF)
    r1 = v - _f32(hi)
    mid = r1.astype(_BF)
    lo = (r1 - _f32(mid)).astype(_BF)
    return hi, mid, lo


def _merge3(g):
    return (g[:, 0:3] + g[:, 3:6]) + g[:, 6:9]


def _knn_body(vb_ref, vaT_ref, o_ref, c_ref, *, V, KPW):
    vb = vb_ref[0]
    vaT = vaT_ref[0]
    x0, x1, x2 = vaT[0:1, :], vaT[1:2, :], vaT[2:3, :]
    qa = x0 * x0 + x1 * x1 + x2 * x2
    b0, b1, b2 = vb[:, 0:1], vb[:, 1:2], vb[:, 2:3]
    qb = b0 * b0 + b1 * b1 + b2 * b2
    inner = jnp.dot(vb, vaT, preferred_element_type=jnp.float32)
    d = (-2.0 * inner + qa) + qb
    iot = jax.lax.broadcasted_iota(jnp.int32, d.shape, 1).astype(jnp.float32)
    kio = jax.lax.broadcasted_iota(jnp.int32, (1, KPW), 1)
    acc = jnp.zeros((vb.shape[0], KPW), jnp.int32)
    fV = jnp.float32(V)
    for k in range(KSEL):
        m = jnp.min(d, axis=1, keepdims=True)
        am = jnp.min(jnp.where(d <= m, iot, fV), axis=1, keepdims=True)
        acc = jnp.where(kio == k, am.astype(jnp.int32), acc)
        d = jnp.where(iot == am, jnp.float32(jnp.inf), d)
    o_ref[0] = acc
    hi, mid, lo = _split3(vb)
    pad = jnp.zeros((vb.shape[0], TC0 - 9), _BF)
    c_ref[0] = jnp.concatenate([hi, mid, lo, pad], axis=1)


def _knn(verts, V, BR):
    vaT = jnp.swapaxes(verts, 1, 2)
    grid = (B, V // BR)
    return pl.pallas_call(
        functools.partial(_knn_body, V=V, KPW=KP),
        grid=grid,
        in_specs=[
            pl.BlockSpec((1, BR, 3), lambda b, i: (b, i, 0)),
            pl.BlockSpec((1, 3, V), lambda b, i: (b, 0, 0)),
        ],
        out_specs=[
            pl.BlockSpec((1, BR, KP), lambda b, i: (b, i, 0)),
            pl.BlockSpec((1, BR, TC0), lambda b, i: (b, i, 0)),
        ],
        out_shape=[
            jax.ShapeDtypeStruct((B, V, KP), jnp.int32),
            jax.ShapeDtypeStruct((B, V, TC0), _BF),
        ],
    )(verts, vaT)


def _gather(iotc, ta8, idx, D, G):
    if G == 1:
        ohc = (iotc == idx).astype(jnp.float32)
        return jnp.dot(ohc, _f32(ta8), preferred_element_type=jnp.float32)
    ohc = (iotc == (idx >> 3)).astype(_BF)
    g8 = jnp.dot(ohc, ta8, preferred_element_type=jnp.float32)
    ilo = idx & (G - 1)
    g = g8[:, 0:D] * (ilo == 0).astype(jnp.float32)
    for j in range(1, G):
        g = g + g8[:, j * D:(j + 1) * D] * (ilo == j).astype(jnp.float32)
    return g


def _surface_body(vb_ref, ct_ref, nb_ref, dir_ref, o_ref, *, V, G, Cout,
                  DoutP):
    vb = vb_ref[0]
    ct = ct_ref[0]
    nb = nb_ref[0]
    dr = dir_ref[...]
    nrm = jnp.sqrt(dr[0:1] * dr[0:1] + dr[1:2] * dr[1:2] + dr[2:3] * dr[2:3])
    sdn = dr / jnp.maximum(nrm, 1e-12)
    BR = vb.shape[0]
    iotc = jax.lax.broadcasted_iota(jnp.int32, (BR, V // G), 1)
    accs = [jnp.full((BR, Cout), -jnp.inf, jnp.float32) for _ in range(4)]
    for n in range(NN):
        g = _gather(iotc, ct, nb[:, n + 1:n + 2], TC0, G)
        dx = _merge3(g) - vb
        dn = jnp.sqrt(dx[:, 0:1] ** 2 + dx[:, 1:2] ** 2 + dx[:, 2:3] ** 2)
        dx = dx / jnp.maximum(dn, 1e-12)
        theta = jax.nn.relu(jnp.dot(dx, sdn, preferred_element_type=jnp.float32))
        accs[n % 4] = jnp.maximum(accs[n % 4], theta)
    acc = jnp.maximum(jnp.maximum(accs[0], accs[1]),
                      jnp.maximum(accs[2], accs[3]))
    hi, mid, lo = _split3(vb)
    pad = jnp.zeros((BR, DoutP - 9 - Cout), _BF)
    o_ref[0] = jnp.concatenate([hi, mid, lo, acc.astype(_BF), pad], axis=1)


def _surface(verts, ctab, nbg, dir0, BR, G=1):
    grid = (B, V1 // BR)
    ctab8 = ctab.reshape(B, V1 // G, G * TC0)
    return pl.pallas_call(
        functools.partial(_surface_body, V=V1, G=G, Cout=32, DoutP=T0),
        grid=grid,
        in_specs=[
            pl.BlockSpec((1, BR, 3), lambda b, i: (b, i, 0)),
            pl.BlockSpec((1, V1 // G, G * TC0), lambda b, i: (b, 0, 0)),
            pl.BlockSpec((1, BR, KP), lambda b, i: (b, i, 0)),
            pl.BlockSpec((3, 32), lambda b, i: (0, 0)),
        ],
        out_specs=pl.BlockSpec((1, BR, T0), lambda b, i: (b, i, 0)),
        out_shape=jax.ShapeDtypeStruct((B, V1, T0), _BF),
    )(verts, ctab8, nbg, dir0)


def _conv_body(tb_ref, ta_ref, nb_ref, dir_ref, w_ref, b_ref, o_ref, *,
               V, G, Cin, Cout, DinP, DoutP, relu):
    tb = tb_ref[0]
    ta = ta_ref[0]
    nb = nb_ref[0]
    dr = dir_ref[...]
    w = w_ref[...]
    bias = b_ref[...]
    nrm = jnp.sqrt(dr[0:1] * dr[0:1] + dr[1:2] * dr[1:2] + dr[2:3] * dr[2:3])
    sdn = dr / jnp.maximum(nrm, 1e-12)
    vb = _merge3(_f32(tb[:, 0:9]))
    fm = tb[:, 9:9 + Cin]
    center = (jnp.dot(fm, w[:, :Cout], preferred_element_type=jnp.float32)
              + bias[:, :Cout])
    BR = tb.shape[0]
    iotc = jax.lax.broadcasted_iota(jnp.int32, (BR, V // G), 1)
    accs = [jnp.full((BR, Cout), -jnp.inf, jnp.float32) for _ in range(4)]
    for n in range(NN):
        g = _gather(iotc, ta, nb[:, n + 1:n + 2], DinP, G)
        dx = _merge3(g) - vb
        dn = jnp.sqrt(dx[:, 0:1] ** 2 + dx[:, 1:2] ** 2 + dx[:, 2:3] ** 2)
        dx = dx / jnp.maximum(dn, 1e-12)
        theta = jax.nn.relu(jnp.dot(dx, sdn, preferred_element_type=jnp.float32))
        fs = (jnp.dot(g[:, 9:9 + Cin].astype(_BF), w[:, Cout:],
                      preferred_element_type=jnp.float32) + bias[:, Cout:])
        accs[n % 4] = jnp.maximum(accs[n % 4], theta * fs)
    acc = jnp.maximum(jnp.maximum(accs[0], accs[1]),
                      jnp.maximum(accs[2], accs[3]))
    out = center + acc
    if relu:
        out = jax.nn.relu(out)
    pad = jnp.zeros((BR, DoutP - 9 - Cout), _BF)
    o_ref[0] = jnp.concatenate([tb[:, 0:9], out.astype(_BF), pad], axis=1)


def _conv(table, nbg, dirw, w, bias, V, BR, Cin, Cout, DinP, DoutP,
          relu=True, G=1):
    grid = (B, V // BR)
    ta8 = table.reshape(B, V // G, G * DinP)
    return pl.pallas_call(
        functools.partial(_conv_body, V=V, G=G, Cin=Cin, Cout=Cout,
                          DinP=DinP, DoutP=DoutP, relu=relu),
        grid=grid,
        in_specs=[
            pl.BlockSpec((1, BR, DinP), lambda b, i: (b, i, 0)),
            pl.BlockSpec((1, V // G, G * DinP), lambda b, i: (b, 0, 0)),
            pl.BlockSpec((1, BR, KP), lambda b, i: (b, i, 0)),
            pl.BlockSpec((3, Cout), lambda b, i: (0, 0)),
            pl.BlockSpec((Cin, 2 * Cout), lambda b, i: (0, 0)),
            pl.BlockSpec((1, 2 * Cout), lambda b, i: (0, 0)),
        ],
        out_specs=pl.BlockSpec((1, BR, DoutP), lambda b, i: (b, i, 0)),
        out_shape=jax.ShapeDtypeStruct((B, V, DoutP), _BF),
    )(table, ta8, nbg, dirw, w.astype(_BF), bias.reshape(1, -1))


def _pool_body(ta_ref, nb_ref, s_ref, o_ref, *, V, C, DinP, DoutP):
    ta = ta_ref[0]
    nb = nb_ref[0]
    s = s_ref[0]
    Np = s.shape[0]
    iot = jax.lax.broadcasted_iota(jnp.int32, (Np, V), 1)
    nb4 = nb[:, 1:5]
    nbh = (nb4 >> 8).astype(jnp.float32).astype(_BF)
    nbl = (nb4 & 255).astype(jnp.float32).astype(_BF)
    cat = jnp.concatenate([ta, nbh, nbl], axis=1)
    ohs = (iot == s).astype(_BF)
    gs = jnp.dot(ohs, cat, preferred_element_type=jnp.float32)
    idx4 = (gs[:, DinP:DinP + 4] * 256.0
            + gs[:, DinP + 4:DinP + 8]).astype(jnp.int32)
    acc = jnp.full((Np, C), -jnp.inf, jnp.float32)
    for j in range(4):
        oh = (iot == idx4[:, j:j + 1]).astype(_BF)
        g = jnp.dot(oh, ta, preferred_element_type=jnp.float32)
        acc = jnp.maximum(acc, g[:, 9:9 + C])
    pad = jnp.zeros((Np, DoutP - 9 - C), _BF)
    o_ref[0] = jnp.concatenate([gs[:, 0:9].astype(_BF), acc.astype(_BF), pad],
                               axis=1)


def _pool(table, nbg, samp, V, Np, C, DinP, DoutP):
    samp3 = jnp.broadcast_to(samp.reshape(1, Np, 1), (B, Np, 1))
    return pl.pallas_call(
        functools.partial(_pool_body, V=V, C=C, DinP=DinP, DoutP=DoutP),
        grid=(B,),
        in_specs=[
            pl.BlockSpec((1, V, DinP), lambda b: (b, 0, 0)),
            pl.BlockSpec((1, V, KP), lambda b: (b, 0, 0)),
            pl.BlockSpec((1, Np, 1), lambda b: (b, 0, 0)),
        ],
        out_specs=pl.BlockSpec((1, Np, DoutP), lambda b: (b, 0, 0)),
        out_shape=jax.ShapeDtypeStruct((B, Np, DoutP), _BF),
    )(table, nbg, samp3)


def _conv4_body(ta_ref, nb_ref, dir_ref, w_ref, b_ref, o_ref, *,
                V, Cin, Cout):
    ta = ta_ref[0]
    nb = nb_ref[0]
    dr = dir_ref[...]
    w = w_ref[...]
    bias = b_ref[...]
    nrm = jnp.sqrt(dr[0:1] * dr[0:1] + dr[1:2] * dr[1:2] + dr[2:3] * dr[2:3])
    sdn = dr / jnp.maximum(nrm, 1e-12)
    vb = _merge3(_f32(ta[:, 0:9]))
    fm = ta[:, 9:9 + Cin]
    center = (jnp.dot(fm, w[:, :Cout], preferred_element_type=jnp.float32)
              + bias[:, :Cout])
    iot = jax.lax.broadcasted_iota(jnp.int32, (V, V), 1)
    accs = [jnp.full((V, Cout), -jnp.inf, jnp.float32) for _ in range(4)]
    taf = _f32(ta)
    for n in range(NN):
        oh = (iot == nb[:, n + 1:n + 2]).astype(jnp.float32)
        g = jnp.dot(oh, taf, preferred_element_type=jnp.float32)
        dx = _merge3(g) - vb
        dn = jnp.sqrt(dx[:, 0:1] ** 2 + dx[:, 1:2] ** 2 + dx[:, 2:3] ** 2)
        dx = dx / jnp.maximum(dn, 1e-12)
        theta = jax.nn.relu(jnp.dot(dx, sdn, preferred_element_type=jnp.float32))
        fs = (jnp.dot(g[:, 9:9 + Cin].astype(_BF), w[:, Cout:],
                      preferred_element_type=jnp.float32) + bias[:, Cout:])
        accs[n % 4] = jnp.maximum(accs[n % 4], theta * fs)
    acc = jnp.maximum(jnp.maximum(accs[0], accs[1]),
                      jnp.maximum(accs[2], accs[3]))
    fm4 = center + acc
    o_ref[...] = jnp.max(fm4, axis=0, keepdims=True)[None]


def _conv4_emb(table, nbg, dirw, w, bias):
    return pl.pallas_call(
        functools.partial(_conv4_body, V=V3, Cin=256, Cout=512),
        grid=(B,),
        in_specs=[
            pl.BlockSpec((1, V3, T3), lambda b: (b, 0, 0)),
            pl.BlockSpec((1, V3, KP), lambda b: (b, 0, 0)),
            pl.BlockSpec((3, 512), lambda b: (0, 0)),
            pl.BlockSpec((256, 1024), lambda b: (0, 0)),
            pl.BlockSpec((1, 1024), lambda b: (0, 0)),
        ],
        out_specs=pl.BlockSpec((1, 1, 512), lambda b: (b, 0, 0)),
        out_shape=jax.ShapeDtypeStruct((B, 1, 512), jnp.float32),
    )(table, nbg, dirw, w.astype(_BF), bias.reshape(1, -1)).reshape(B, 512)


def _fc_body(e_ref, w1_ref, b1_ref, w2_ref, b2_ref, w3_ref, b3_ref, o_ref):
    emb = e_ref[...]
    h1 = jax.nn.relu(jnp.dot(emb, w1_ref[...],
                             preferred_element_type=jnp.float32) + b1_ref[...])
    h2 = jax.nn.relu(jnp.dot(h1, w2_ref[...],
                             preferred_element_type=jnp.float32) + b2_ref[...])
    o_ref[...] = (jnp.dot(h2, w3_ref[...], preferred_element_type=jnp.float32)
                  + b3_ref[...])


def _fc_head(emb, fc1_w, fc1_b, fc2_w, fc2_b, fc3_w, fc3_b):
    NB = 512
    grid = (3072 // NB,)
    return pl.pallas_call(
        _fc_body,
        grid=grid,
        in_specs=[
            pl.BlockSpec((B, 512), lambda i: (0, 0)),
            pl.BlockSpec((512, 512), lambda i: (0, 0)),
            pl.BlockSpec((1, 512), lambda i: (0, 0)),
            pl.BlockSpec((512, 1024), lambda i: (0, 0)),
            pl.BlockSpec((1, 1024), lambda i: (0, 0)),
            pl.BlockSpec((1024, NB), lambda i: (0, i)),
            pl.BlockSpec((1, NB), lambda i: (0, i)),
        ],
        out_specs=pl.BlockSpec((B, NB), lambda i: (0, i)),
        out_shape=jax.ShapeDtypeStruct((B, 3072), jnp.float32),
    )(emb, fc1_w, fc1_b.reshape(1, -1), fc2_w, fc2_b.reshape(1, -1),
      fc3_w, fc3_b.reshape(1, -1))


def _coords(table):
    return ((_f32(table[:, :, 0:3]) + _f32(table[:, :, 3:6]))
            + _f32(table[:, :, 6:9]))


def kernel(in_pc, dir0, w1, b1, dir1, w2, b2, dir2, w3, b3, dir3, w4, b4,
           dir4, fc1_w, fc1_b, fc2_w, fc2_b, fc3_w, fc3_b):
    s1 = jax.random.permutation(jax.random.key(101), V1)[: V1 // 4]
    s2 = jax.random.permutation(jax.random.key(102), V2)[: V2 // 4]
    s1 = s1.astype(jnp.int32)
    s2 = s2.astype(jnp.int32)

    nbg1, ctab1 = _knn(in_pc, V1, 256)
    t0 = _surface(in_pc, ctab1, nbg1, dir0, 256)
    t1 = _conv(t0, nbg1, dir1, w1, b1, V1, 256, 32, 64, T0, T1)
    t1p = _pool(t1, nbg1, s1, V1, V2, 64, T1, T1)

    v1 = _coords(t1p)
    nbg2, _ = _knn(v1, V2, 512)
    t2 = _conv(t1p, nbg2, dir2, w2, b2, V2, 512, 64, 128, T1, T2)
    t3 = _conv(t2, nbg2, dir3, w3, b3, V2, 512, 128, 256, T2, T3)
    t3p = _pool(t3, nbg2, s2, V2, V3, 256, T3, T3)

    v2 = _coords(t3p)
    nbg3, _ = _knn(v2, V3, 128)
    emb = _conv4_emb(t3p, nbg3, dir4, w4, b4)

    out3 = _fc_head(emb, fc1_w, fc1_b, fc2_w, fc2_b, fc3_w, fc3_b)
    return emb, out3.reshape(B, 1024, 3)


# speedup vs baseline: 1.2351x; 1.2351x over previous
import functools
import math

import jax
import jax.numpy as jnp
import numpy as np
from jax.experimental import pallas as pl
from jax.experimental.pallas import tpu as pltpu

B = 8
V1, V2, V3 = 2048, 512, 128
NN = 20
KSEL = 21
KP = 24

TC0 = 16
T0 = 48
T1 = 80
T2 = 144
T3 = 272

_BF = jnp.bfloat16


def _f32(x):
    return x.astype(jnp.float32)


def _split3(v):
    hi = v.astype(_BF)
    r1 = v - _f32(hi)
    mid = r1.astype(_BF)
    lo = (r1 - _f32(mid)).astype(_BF)
    return hi, mid, lo


def _merge3(g):
    return (g[:, 0:3] + g[:, 3:6]) + g[:, 6:9]


def _knn_body(vb_ref, vaT_ref, o_ref, c_ref, *, V, KPW):
    vb = vb_ref[0]
    vaT = vaT_ref[0]
    x0, x1, x2 = vaT[0:1, :], vaT[1:2, :], vaT[2:3, :]
    qa = x0 * x0 + x1 * x1 + x2 * x2
    b0, b1, b2 = vb[:, 0:1], vb[:, 1:2], vb[:, 2:3]
    qb = b0 * b0 + b1 * b1 + b2 * b2
    inner = jnp.dot(vb, vaT, preferred_element_type=jnp.float32)
    d = (-2.0 * inner + qa) + qb
    iot = jax.lax.broadcasted_iota(jnp.int32, d.shape, 1).astype(jnp.float32)
    kio = jax.lax.broadcasted_iota(jnp.int32, (1, KPW), 1)
    acc = jnp.zeros((vb.shape[0], KPW), jnp.int32)
    fV = jnp.float32(V)
    for k in range(KSEL):
        m = jnp.min(d, axis=1, keepdims=True)
        am = jnp.min(jnp.where(d <= m, iot, fV), axis=1, keepdims=True)
        acc = jnp.where(kio == k, am.astype(jnp.int32), acc)
        d = jnp.where(iot == am, jnp.float32(jnp.inf), d)
    o_ref[0] = acc
    hi, mid, lo = _split3(vb)
    pad = jnp.zeros((vb.shape[0], TC0 - 9), _BF)
    c_ref[0] = jnp.concatenate([hi, mid, lo, pad], axis=1)


def _knn(verts, V, BR):
    vaT = jnp.swapaxes(verts, 1, 2)
    grid = (B, V // BR)
    return pl.pallas_call(
        functools.partial(_knn_body, V=V, KPW=KP),
        grid=grid,
        in_specs=[
            pl.BlockSpec((1, BR, 3), lambda b, i: (b, i, 0)),
            pl.BlockSpec((1, 3, V), lambda b, i: (b, 0, 0)),
        ],
        out_specs=[
            pl.BlockSpec((1, BR, KP), lambda b, i: (b, i, 0)),
            pl.BlockSpec((1, BR, TC0), lambda b, i: (b, i, 0)),
        ],
        out_shape=[
            jax.ShapeDtypeStruct((B, V, KP), jnp.int32),
            jax.ShapeDtypeStruct((B, V, TC0), _BF),
        ],
    )(verts, vaT)


def _gather(iotc, ta8, idx, D, G):
    if G == 1:
        ohc = (iotc == idx).astype(_BF)
        return jnp.dot(ohc, ta8, preferred_element_type=jnp.float32)
    ohc = (iotc == (idx >> 3)).astype(_BF)
    g8 = jnp.dot(ohc, ta8, preferred_element_type=jnp.float32)
    ilo = idx & (G - 1)
    g = g8[:, 0:D] * (ilo == 0).astype(jnp.float32)
    for j in range(1, G):
        g = g + g8[:, j * D:(j + 1) * D] * (ilo == j).astype(jnp.float32)
    return g


def _surface_body(vb_ref, ct_ref, nb_ref, dir_ref, o_ref, *, V, G, Cout,
                  DoutP):
    vb = vb_ref[0]
    ct = ct_ref[0]
    nb = nb_ref[0]
    dr = dir_ref[...]
    nrm = jnp.sqrt(dr[0:1] * dr[0:1] + dr[1:2] * dr[1:2] + dr[2:3] * dr[2:3])
    sdn = dr / jnp.maximum(nrm, 1e-12)
    BR = vb.shape[0]
    iotc = jax.lax.broadcasted_iota(jnp.int32, (BR, V // G), 1)
    accs = [jnp.full((BR, Cout), -jnp.inf, jnp.float32) for _ in range(4)]
    for n in range(NN):
        g = _gather(iotc, ct, nb[:, n + 1:n + 2], TC0, G)
        dx = _merge3(g) - vb
        dn = jnp.sqrt(dx[:, 0:1] ** 2 + dx[:, 1:2] ** 2 + dx[:, 2:3] ** 2)
        dx = dx / jnp.maximum(dn, 1e-12)
        theta = jax.nn.relu(jnp.dot(dx, sdn, preferred_element_type=jnp.float32))
        accs[n % 4] = jnp.maximum(accs[n % 4], theta)
    acc = jnp.maximum(jnp.maximum(accs[0], accs[1]),
                      jnp.maximum(accs[2], accs[3]))
    hi, mid, lo = _split3(vb)
    pad = jnp.zeros((BR, DoutP - 9 - Cout), _BF)
    o_ref[0] = jnp.concatenate([hi, mid, lo, acc.astype(_BF), pad], axis=1)


def _surface(verts, ctab, nbg, dir0, BR, G=1):
    grid = (B, V1 // BR)
    ctab8 = ctab.reshape(B, V1 // G, G * TC0)
    return pl.pallas_call(
        functools.partial(_surface_body, V=V1, G=G, Cout=32, DoutP=T0),
        grid=grid,
        in_specs=[
            pl.BlockSpec((1, BR, 3), lambda b, i: (b, i, 0)),
            pl.BlockSpec((1, V1 // G, G * TC0), lambda b, i: (b, 0, 0)),
            pl.BlockSpec((1, BR, KP), lambda b, i: (b, i, 0)),
            pl.BlockSpec((3, 32), lambda b, i: (0, 0)),
        ],
        out_specs=pl.BlockSpec((1, BR, T0), lambda b, i: (b, i, 0)),
        out_shape=jax.ShapeDtypeStruct((B, V1, T0), _BF),
    )(verts, ctab8, nbg, dir0)


def _conv_body(tb_ref, ta_ref, nb_ref, dir_ref, w_ref, b_ref, o_ref, *,
               V, G, Cin, Cout, DinP, DoutP, relu):
    tb = tb_ref[0]
    ta = ta_ref[0]
    nb = nb_ref[0]
    dr = dir_ref[...]
    w = w_ref[...]
    bias = b_ref[...]
    nrm = jnp.sqrt(dr[0:1] * dr[0:1] + dr[1:2] * dr[1:2] + dr[2:3] * dr[2:3])
    sdn = dr / jnp.maximum(nrm, 1e-12)
    vb = _merge3(_f32(tb[:, 0:9]))
    fm = tb[:, 9:9 + Cin]
    center = (jnp.dot(fm, w[:, :Cout], preferred_element_type=jnp.float32)
              + bias[:, :Cout])
    BR = tb.shape[0]
    iotc = jax.lax.broadcasted_iota(jnp.int32, (BR, V // G), 1)
    accs = [jnp.full((BR, Cout), -jnp.inf, jnp.float32) for _ in range(4)]
    for n in range(NN):
        g = _gather(iotc, ta, nb[:, n + 1:n + 2], DinP, G)
        dx = _merge3(g) - vb
        dn = jnp.sqrt(dx[:, 0:1] ** 2 + dx[:, 1:2] ** 2 + dx[:, 2:3] ** 2)
        dx = dx / jnp.maximum(dn, 1e-12)
        theta = jax.nn.relu(jnp.dot(dx, sdn, preferred_element_type=jnp.float32))
        fs = (jnp.dot(g[:, 9:9 + Cin].astype(_BF), w[:, Cout:],
                      preferred_element_type=jnp.float32) + bias[:, Cout:])
        accs[n % 4] = jnp.maximum(accs[n % 4], theta * fs)
    acc = jnp.maximum(jnp.maximum(accs[0], accs[1]),
                      jnp.maximum(accs[2], accs[3]))
    out = center + acc
    if relu:
        out = jax.nn.relu(out)
    pad = jnp.zeros((BR, DoutP - 9 - Cout), _BF)
    o_ref[0] = jnp.concatenate([tb[:, 0:9], out.astype(_BF), pad], axis=1)


def _conv(table, nbg, dirw, w, bias, V, BR, Cin, Cout, DinP, DoutP,
          relu=True, G=1):
    grid = (B, V // BR)
    ta8 = table.reshape(B, V // G, G * DinP)
    return pl.pallas_call(
        functools.partial(_conv_body, V=V, G=G, Cin=Cin, Cout=Cout,
                          DinP=DinP, DoutP=DoutP, relu=relu),
        grid=grid,
        in_specs=[
            pl.BlockSpec((1, BR, DinP), lambda b, i: (b, i, 0)),
            pl.BlockSpec((1, V // G, G * DinP), lambda b, i: (b, 0, 0)),
            pl.BlockSpec((1, BR, KP), lambda b, i: (b, i, 0)),
            pl.BlockSpec((3, Cout), lambda b, i: (0, 0)),
            pl.BlockSpec((Cin, 2 * Cout), lambda b, i: (0, 0)),
            pl.BlockSpec((1, 2 * Cout), lambda b, i: (0, 0)),
        ],
        out_specs=pl.BlockSpec((1, BR, DoutP), lambda b, i: (b, i, 0)),
        out_shape=jax.ShapeDtypeStruct((B, V, DoutP), _BF),
    )(table, ta8, nbg, dirw, w.astype(_BF), bias.reshape(1, -1))


def _pool_body(ta_ref, nb_ref, s_ref, o_ref, *, V, C, DinP, DoutP):
    ta = ta_ref[0]
    nb = nb_ref[0]
    s = s_ref[0]
    Np = s.shape[0]
    iot = jax.lax.broadcasted_iota(jnp.int32, (Np, V), 1)
    nb4 = nb[:, 1:5]
    nbh = (nb4 >> 8).astype(jnp.float32).astype(_BF)
    nbl = (nb4 & 255).astype(jnp.float32).astype(_BF)
    cat = jnp.concatenate([ta, nbh, nbl], axis=1)
    ohs = (iot == s).astype(_BF)
    gs = jnp.dot(ohs, cat, preferred_element_type=jnp.float32)
    idx4 = (gs[:, DinP:DinP + 4] * 256.0
            + gs[:, DinP + 4:DinP + 8]).astype(jnp.int32)
    acc = jnp.full((Np, C), -jnp.inf, jnp.float32)
    for j in range(4):
        oh = (iot == idx4[:, j:j + 1]).astype(_BF)
        g = jnp.dot(oh, ta, preferred_element_type=jnp.float32)
        acc = jnp.maximum(acc, g[:, 9:9 + C])
    pad = jnp.zeros((Np, DoutP - 9 - C), _BF)
    o_ref[0] = jnp.concatenate([gs[:, 0:9].astype(_BF), acc.astype(_BF), pad],
                               axis=1)


def _pool(table, nbg, samp, V, Np, C, DinP, DoutP):
    samp3 = jnp.broadcast_to(samp.reshape(1, Np, 1), (B, Np, 1))
    return pl.pallas_call(
        functools.partial(_pool_body, V=V, C=C, DinP=DinP, DoutP=DoutP),
        grid=(B,),
        in_specs=[
            pl.BlockSpec((1, V, DinP), lambda b: (b, 0, 0)),
            pl.BlockSpec((1, V, KP), lambda b: (b, 0, 0)),
            pl.BlockSpec((1, Np, 1), lambda b: (b, 0, 0)),
        ],
        out_specs=pl.BlockSpec((1, Np, DoutP), lambda b: (b, 0, 0)),
        out_shape=jax.ShapeDtypeStruct((B, Np, DoutP), _BF),
    )(table, nbg, samp3)


def _conv4_body(ta_ref, nb_ref, dir_ref, w_ref, b_ref, o_ref, *,
                V, Cin, Cout):
    ta = ta_ref[0]
    nb = nb_ref[0]
    dr = dir_ref[...]
    w = w_ref[...]
    bias = b_ref[...]
    nrm = jnp.sqrt(dr[0:1] * dr[0:1] + dr[1:2] * dr[1:2] + dr[2:3] * dr[2:3])
    sdn = dr / jnp.maximum(nrm, 1e-12)
    vb = _merge3(_f32(ta[:, 0:9]))
    fm = ta[:, 9:9 + Cin]
    center = (jnp.dot(fm, w[:, :Cout], preferred_element_type=jnp.float32)
              + bias[:, :Cout])
    iot = jax.lax.broadcasted_iota(jnp.int32, (V, V), 1)
    accs = [jnp.full((V, Cout), -jnp.inf, jnp.float32) for _ in range(4)]
    for n in range(NN):
        oh = (iot == nb[:, n + 1:n + 2]).astype(_BF)
        g = jnp.dot(oh, ta, preferred_element_type=jnp.float32)
        dx = _merge3(g) - vb
        dn = jnp.sqrt(dx[:, 0:1] ** 2 + dx[:, 1:2] ** 2 + dx[:, 2:3] ** 2)
        dx = dx / jnp.maximum(dn, 1e-12)
        theta = jax.nn.relu(jnp.dot(dx, sdn, preferred_element_type=jnp.float32))
        fs = (jnp.dot(g[:, 9:9 + Cin].astype(_BF), w[:, Cout:],
                      preferred_element_type=jnp.float32) + bias[:, Cout:])
        accs[n % 4] = jnp.maximum(accs[n % 4], theta * fs)
    acc = jnp.maximum(jnp.maximum(accs[0], accs[1]),
                      jnp.maximum(accs[2], accs[3]))
    fm4 = center + acc
    o_ref[...] = jnp.max(fm4, axis=0, keepdims=True)[None]


def _conv4_emb(table, nbg, dirw, w, bias):
    return pl.pallas_call(
        functools.partial(_conv4_body, V=V3, Cin=256, Cout=512),
        grid=(B,),
        in_specs=[
            pl.BlockSpec((1, V3, T3), lambda b: (b, 0, 0)),
            pl.BlockSpec((1, V3, KP), lambda b: (b, 0, 0)),
            pl.BlockSpec((3, 512), lambda b: (0, 0)),
            pl.BlockSpec((256, 1024), lambda b: (0, 0)),
            pl.BlockSpec((1, 1024), lambda b: (0, 0)),
        ],
        out_specs=pl.BlockSpec((1, 1, 512), lambda b: (b, 0, 0)),
        out_shape=jax.ShapeDtypeStruct((B, 1, 512), jnp.float32),
    )(table, nbg, dirw, w.astype(_BF), bias.reshape(1, -1)).reshape(B, 512)


def _fc_body(e_ref, w1_ref, b1_ref, w2_ref, b2_ref, w3_ref, b3_ref, o_ref):
    emb = e_ref[...]
    h1 = jax.nn.relu(jnp.dot(emb, w1_ref[...],
                             preferred_element_type=jnp.float32) + b1_ref[...])
    h2 = jax.nn.relu(jnp.dot(h1, w2_ref[...],
                             preferred_element_type=jnp.float32) + b2_ref[...])
    o_ref[...] = (jnp.dot(h2, w3_ref[...], preferred_element_type=jnp.float32)
                  + b3_ref[...])


def _fc_head(emb, fc1_w, fc1_b, fc2_w, fc2_b, fc3_w, fc3_b):
    NB = 512
    grid = (3072 // NB,)
    return pl.pallas_call(
        _fc_body,
        grid=grid,
        in_specs=[
            pl.BlockSpec((B, 512), lambda i: (0, 0)),
            pl.BlockSpec((512, 512), lambda i: (0, 0)),
            pl.BlockSpec((1, 512), lambda i: (0, 0)),
            pl.BlockSpec((512, 1024), lambda i: (0, 0)),
            pl.BlockSpec((1, 1024), lambda i: (0, 0)),
            pl.BlockSpec((1024, NB), lambda i: (0, i)),
            pl.BlockSpec((1, NB), lambda i: (0, i)),
        ],
        out_specs=pl.BlockSpec((B, NB), lambda i: (0, i)),
        out_shape=jax.ShapeDtypeStruct((B, 3072), jnp.float32),
    )(emb, fc1_w, fc1_b.reshape(1, -1), fc2_w, fc2_b.reshape(1, -1),
      fc3_w, fc3_b.reshape(1, -1))


def _coords(table):
    return ((_f32(table[:, :, 0:3]) + _f32(table[:, :, 3:6]))
            + _f32(table[:, :, 6:9]))


def kernel(in_pc, dir0, w1, b1, dir1, w2, b2, dir2, w3, b3, dir3, w4, b4,
           dir4, fc1_w, fc1_b, fc2_w, fc2_b, fc3_w, fc3_b):
    s1 = jax.random.permutation(jax.random.key(101), V1)[: V1 // 4]
    s2 = jax.random.permutation(jax.random.key(102), V2)[: V2 // 4]
    s1 = s1.astype(jnp.int32)
    s2 = s2.astype(jnp.int32)

    nbg1, ctab1 = _knn(in_pc, V1, 256)
    t0 = _surface(in_pc, ctab1, nbg1, dir0, 256)
    t1 = _conv(t0, nbg1, dir1, w1, b1, V1, 256, 32, 64, T0, T1)
    t1p = _pool(t1, nbg1, s1, V1, V2, 64, T1, T1)

    v1 = _coords(t1p)
    nbg2, _ = _knn(v1, V2, 512)
    t2 = _conv(t1p, nbg2, dir2, w2, b2, V2, 512, 64, 128, T1, T2)
    t3 = _conv(t2, nbg2, dir3, w3, b3, V2, 512, 128, 256, T2, T3)
    t3p = _pool(t3, nbg2, s2, V2, V3, 256, T3, T3)

    v2 = _coords(t3p)
    nbg3, _ = _knn(v2, V3, 128)
    emb = _conv4_emb(t3p, nbg3, dir4, w4, b4)

    out3 = _fc_head(emb, fc1_w, fc1_b, fc2_w, fc2_b, fc3_w, fc3_b)
    return emb, out3.reshape(B, 1024, 3)
